# Initial kernel scaffold; baseline (speedup 1.0000x reference)
import jax
import jax.numpy as jnp
from jax.experimental import pallas as pl


def kernel(sparse_0, sparse_1, sparse_2, sparse_3, sparse_4, sparse_5, sparse_6, sparse_7, sparse_8, sparse_9, sparse_10, sparse_11, sparse_12, sparse_13, sparse_14, sparse_15, sparse_16, sparse_17, sparse_18, sparse_19, sparse_20, sparse_21, sparse_22, sparse_23, sparse_24, sparse_25, dense_0, dense_1, dense_2, dense_3, dense_4, dense_5, dense_6, dense_7, dense_8, dense_9, dense_10, dense_11, dense_12, emb_0, emb_1, emb_2, emb_3, emb_4, emb_5, emb_6, emb_7, emb_8, emb_9, emb_10, emb_11, emb_12, emb_13, emb_14, emb_15, emb_16, emb_17, emb_18, emb_19, emb_20, emb_21, emb_22, emb_23, emb_24, emb_25, fc_W, fc_b):
    raise NotImplementedError("write your pallas kernel here")


# speedup vs baseline: 1.1148x; 1.1148x over previous
import functools

import jax
import jax.numpy as jnp
from jax import lax
from jax.experimental import pallas as pl
from jax.experimental.pallas import tpu as pltpu
from jax.experimental.pallas import tpu_sc as plsc

_NSF = 26
_NDF = 13
_BATCH = 16384
_NC = 2
_NSUB = 16
_NW = _NC * _NSUB
_BPW = _BATCH // _NW
_L = 16
_CH = 128
_NCH = _BPW // _CH
_NSL = _BPW // _L


def _sc_body(idx_hbm, dense_hbm, w_hbm, *rest):
    tabs = rest[:_NSF]
    out_hbm = rest[_NSF]
    idx_v, dense_v, w_v, vals_v, acc_v, sem = rest[_NSF + 1:]

    wid = lax.axis_index("s") * _NC + lax.axis_index("c")
    base = wid * _BPW

    pltpu.sync_copy(idx_hbm.at[:, pl.ds(wid * _NCH, _NCH), :], idx_v)
    pltpu.sync_copy(dense_hbm.at[:, pl.ds(base, _BPW)], dense_v)
    pltpu.sync_copy(w_hbm, w_v)

    def chunk_body(lo):
        def body(j, carry):
            cps = [
                pltpu.async_copy(
                    tabs[i].at[idx_v.at[i, j]],
                    vals_v.at[i, pl.ds(j * _CH, _CH)],
                    sem,
                )
                for i in range(lo, lo + 13)
            ]
            for cp in cps:
                cp.wait()
            return carry
        return body

    lax.fori_loop(0, _NCH, chunk_body(0), 0, unroll=False)
    lax.fori_loop(0, _NCH, chunk_body(13), 0, unroll=False)

    def acc_body(s, carry):
        sl = pl.ds(s * _L, _L)
        a = w_v[_NSF + _NDF, :]
        for i in range(_NSF):
            a = a + vals_v[i, sl] * w_v[i, :]
        for j in range(_NDF):
            a = a + dense_v[j, sl] * w_v[_NSF + j, :]
        acc_v[sl] = a
        return carry

    lax.fori_loop(0, _NSL, acc_body, 0, unroll=False)

    pltpu.sync_copy(acc_v, out_hbm.at[pl.ds(base, _BPW)])


_sc_call = pl.kernel(
    _sc_body,
    out_type=jax.ShapeDtypeStruct((_BATCH,), jnp.float32),
    mesh=plsc.VectorSubcoreMesh(core_axis_name="c", subcore_axis_name="s"),
    scratch_types=[
        pltpu.VMEM((_NSF, _NCH, _CH), jnp.int32),
        pltpu.VMEM((_NDF, _BPW), jnp.float32),
        pltpu.VMEM((_NSF + _NDF + 1, _L), jnp.float32),
        pltpu.VMEM((_NSF, _BPW), jnp.float32),
        pltpu.VMEM((_BPW,), jnp.float32),
        pltpu.SemaphoreType.DMA,
    ],
    name="lr_layer_sc",
)


def kernel(sparse_0, sparse_1, sparse_2, sparse_3, sparse_4, sparse_5, sparse_6, sparse_7, sparse_8, sparse_9, sparse_10, sparse_11, sparse_12, sparse_13, sparse_14, sparse_15, sparse_16, sparse_17, sparse_18, sparse_19, sparse_20, sparse_21, sparse_22, sparse_23, sparse_24, sparse_25, dense_0, dense_1, dense_2, dense_3, dense_4, dense_5, dense_6, dense_7, dense_8, dense_9, dense_10, dense_11, dense_12, emb_0, emb_1, emb_2, emb_3, emb_4, emb_5, emb_6, emb_7, emb_8, emb_9, emb_10, emb_11, emb_12, emb_13, emb_14, emb_15, emb_16, emb_17, emb_18, emb_19, emb_20, emb_21, emb_22, emb_23, emb_24, emb_25, fc_W, fc_b):
    sparse = [sparse_0, sparse_1, sparse_2, sparse_3, sparse_4, sparse_5, sparse_6, sparse_7, sparse_8, sparse_9, sparse_10, sparse_11, sparse_12, sparse_13, sparse_14, sparse_15, sparse_16, sparse_17, sparse_18, sparse_19, sparse_20, sparse_21, sparse_22, sparse_23, sparse_24, sparse_25]
    dense = [dense_0, dense_1, dense_2, dense_3, dense_4, dense_5, dense_6, dense_7, dense_8, dense_9, dense_10, dense_11, dense_12]
    tables = [emb_0, emb_1, emb_2, emb_3, emb_4, emb_5, emb_6, emb_7, emb_8, emb_9, emb_10, emb_11, emb_12, emb_13, emb_14, emb_15, emb_16, emb_17, emb_18, emb_19, emb_20, emb_21, emb_22, emb_23, emb_24, emb_25]

    idx = jnp.stack(sparse, axis=0).astype(jnp.int32)
    idx = idx.reshape(_NSF, _BATCH // _CH, _CH)
    dns = jnp.stack(dense, axis=0).astype(jnp.float32)
    w = jnp.concatenate([fc_W.reshape(-1), fc_b.reshape(-1)]).astype(jnp.float32)
    wrep = jnp.broadcast_to(w[:, None], (_NSF + _NDF + 1, _L))
    flat_tabs = [t.reshape(-1) for t in tables]

    out = _sc_call(idx, dns, wrep, *flat_tabs)
    return out.reshape(_BATCH, 1)


# speedup vs baseline: 22.1706x; 19.8879x over previous
import functools

import jax
import jax.numpy as jnp
from jax import lax
from jax.experimental import pallas as pl
from jax.experimental.pallas import tpu as pltpu
from jax.experimental.pallas import tpu_sc as plsc

_NSF = 26
_NDF = 13
_BATCH = 16384
_NC = 2
_NSUB = 16
_NW = _NC * _NSUB
_BPW = _BATCH // _NW
_L = 16
_CH = 128
_NCH = _BPW // _CH
_NSL = _BPW // _L


def _sc_body(idx_hbm, dense_hbm, w_hbm, *rest):
    tabs = rest[:_NSF]
    out_hbm = rest[_NSF]
    idx_v, dense_v, w_v, vals_v, acc_v, sem = rest[_NSF + 1:]

    wid = lax.axis_index("s") * _NC + lax.axis_index("c")
    base = wid * _BPW

    pltpu.sync_copy(idx_hbm.at[:, pl.ds(wid * _NCH, _NCH), :], idx_v)
    pltpu.sync_copy(dense_hbm.at[:, pl.ds(base, _BPW)], dense_v)
    pltpu.sync_copy(w_hbm, w_v)

    def chunk_body(lo):
        def body(j, carry):
            cps = [
                pltpu.async_copy(
                    tabs[i].at[idx_v.at[i, pl.ds(j, 1), :]],
                    vals_v.at[pl.ds(i, 1), pl.ds(j * _CH, _CH)],
                    sem,
                )
                for i in range(lo, lo + 13)
            ]
            for cp in cps:
                cp.wait()
            return carry
        return body

    lax.fori_loop(0, _NCH, chunk_body(0), 0, unroll=False)
    lax.fori_loop(0, _NCH, chunk_body(13), 0, unroll=False)

    def acc_body(s, carry):
        sl = pl.ds(s * _L, _L)
        a = w_v[_NSF + _NDF, :]
        for i in range(_NSF):
            a = a + vals_v[i, sl] * w_v[i, :]
        for j in range(_NDF):
            a = a + dense_v[j, sl] * w_v[_NSF + j, :]
        acc_v[sl] = a
        return carry

    lax.fori_loop(0, _NSL, acc_body, 0, unroll=False)

    pltpu.sync_copy(acc_v, out_hbm.at[pl.ds(base, _BPW)])


_sc_call = pl.kernel(
    _sc_body,
    out_type=jax.ShapeDtypeStruct((_BATCH,), jnp.float32),
    mesh=plsc.VectorSubcoreMesh(core_axis_name="c", subcore_axis_name="s"),
    scratch_types=[
        pltpu.VMEM((_NSF, _NCH, _CH), jnp.int32),
        pltpu.VMEM((_NDF, _BPW), jnp.float32),
        pltpu.VMEM((_NSF + _NDF + 1, _L), jnp.float32),
        pltpu.VMEM((_NSF, _BPW), jnp.float32),
        pltpu.VMEM((_BPW,), jnp.float32),
        pltpu.SemaphoreType.DMA,
    ],
    name="lr_layer_sc",
)


def kernel(sparse_0, sparse_1, sparse_2, sparse_3, sparse_4, sparse_5, sparse_6, sparse_7, sparse_8, sparse_9, sparse_10, sparse_11, sparse_12, sparse_13, sparse_14, sparse_15, sparse_16, sparse_17, sparse_18, sparse_19, sparse_20, sparse_21, sparse_22, sparse_23, sparse_24, sparse_25, dense_0, dense_1, dense_2, dense_3, dense_4, dense_5, dense_6, dense_7, dense_8, dense_9, dense_10, dense_11, dense_12, emb_0, emb_1, emb_2, emb_3, emb_4, emb_5, emb_6, emb_7, emb_8, emb_9, emb_10, emb_11, emb_12, emb_13, emb_14, emb_15, emb_16, emb_17, emb_18, emb_19, emb_20, emb_21, emb_22, emb_23, emb_24, emb_25, fc_W, fc_b):
    sparse = [sparse_0, sparse_1, sparse_2, sparse_3, sparse_4, sparse_5, sparse_6, sparse_7, sparse_8, sparse_9, sparse_10, sparse_11, sparse_12, sparse_13, sparse_14, sparse_15, sparse_16, sparse_17, sparse_18, sparse_19, sparse_20, sparse_21, sparse_22, sparse_23, sparse_24, sparse_25]
    dense = [dense_0, dense_1, dense_2, dense_3, dense_4, dense_5, dense_6, dense_7, dense_8, dense_9, dense_10, dense_11, dense_12]
    tables = [emb_0, emb_1, emb_2, emb_3, emb_4, emb_5, emb_6, emb_7, emb_8, emb_9, emb_10, emb_11, emb_12, emb_13, emb_14, emb_15, emb_16, emb_17, emb_18, emb_19, emb_20, emb_21, emb_22, emb_23, emb_24, emb_25]

    idx = jnp.stack(sparse, axis=0).astype(jnp.int32)
    idx = idx.reshape(_NSF, _BATCH // _CH, _CH)
    dns = jnp.stack(dense, axis=0).astype(jnp.float32)
    w = jnp.concatenate([fc_W.reshape(-1), fc_b.reshape(-1)]).astype(jnp.float32)
    wrep = jnp.broadcast_to(w[:, None], (_NSF + _NDF + 1, _L))
    wide_tabs = [t.reshape(1, -1) for t in tables]

    out = _sc_call(idx, dns, wrep, *wide_tabs)
    return out.reshape(_BATCH, 1)


# speedup vs baseline: 32.4876x; 1.4653x over previous
import jax
import jax.numpy as jnp
from jax import lax
from jax.experimental import pallas as pl
from jax.experimental.pallas import tpu as pltpu
from jax.experimental.pallas import tpu_sc as plsc

_NSF = 26
_NDF = 13
_BATCH = 16384
_NC = 2
_NSUB = 16
_NW = _NC * _NSUB
_BPW = _BATCH // _NW
_L = 16
_NSL = _BPW // _L


def _sc_body(*refs):
    sparse = refs[:_NSF]
    dense = refs[_NSF:_NSF + _NDF]
    w_hbm = refs[_NSF + _NDF]
    tabs = refs[_NSF + _NDF + 1:_NSF + _NDF + 1 + _NSF]
    out_hbm = refs[_NSF + _NDF + 1 + _NSF]
    scratch = refs[_NSF + _NDF + 2 + _NSF:]
    idx_vs = scratch[:_NSF]
    vals_vs = scratch[_NSF:2 * _NSF]
    (dense_v, w_v, acc_v, sem_in, sem_aux, sem_g) = scratch[2 * _NSF:]

    wid = lax.axis_index("s") * _NC + lax.axis_index("c")
    base = wid * _BPW

    in_cps = [
        pltpu.make_async_copy(
            sparse[i].at[0, pl.ds(base, _BPW)], idx_vs[i], sem_in)
        for i in range(_NSF)
    ]
    aux_cps = [
        pltpu.make_async_copy(
            dense[j].at[:, pl.ds(base, _BPW)], dense_v.at[pl.ds(j, 1), :], sem_aux)
        for j in range(_NDF)
    ]
    aux_cps.append(pltpu.make_async_copy(w_hbm, w_v, sem_aux))
    for cp in in_cps:
        cp.start()
    for cp in aux_cps:
        cp.start()

    g_cps = []
    for i in range(_NSF):
        in_cps[i].wait()
        cp = pltpu.make_async_copy(
            tabs[i].at[0].at[idx_vs[i]], vals_vs[i], sem_g)
        cp.start()
        g_cps.append(cp)

    for cp in aux_cps:
        cp.wait()

    def dense_body(s, carry):
        sl = pl.ds(s * _L, _L)
        a = w_v[_NSF + _NDF, :]
        for j in range(_NDF):
            a = a + dense_v[j, sl] * w_v[_NSF + j, :]
        acc_v[sl] = a
        return carry

    lax.fori_loop(0, _NSL, dense_body, 0, unroll=False)

    for cp in g_cps:
        cp.wait()

    def sparse_body(s, carry):
        sl = pl.ds(s * _L, _L)
        a = acc_v[sl]
        for i in range(_NSF):
            a = a + vals_vs[i][sl] * w_v[i, :]
        acc_v[sl] = a
        return carry

    lax.fori_loop(0, _NSL, sparse_body, 0, unroll=False)

    pltpu.sync_copy(acc_v, out_hbm.at[pl.ds(base, _BPW)])


_sc_call = pl.kernel(
    _sc_body,
    out_type=jax.ShapeDtypeStruct((_BATCH,), jnp.float32),
    mesh=plsc.VectorSubcoreMesh(core_axis_name="c", subcore_axis_name="s"),
    scratch_types=(
        [pltpu.VMEM((_BPW,), jnp.int32) for _ in range(_NSF)]
        + [pltpu.VMEM((_BPW,), jnp.float32) for _ in range(_NSF)]
        + [
            pltpu.VMEM((_NDF, _BPW), jnp.float32),
            pltpu.VMEM((_NSF + _NDF + 1, _L), jnp.float32),
            pltpu.VMEM((_BPW,), jnp.float32),
            pltpu.SemaphoreType.DMA,
            pltpu.SemaphoreType.DMA,
            pltpu.SemaphoreType.DMA,
        ]
    ),
    name="lr_layer_sc",
)


def kernel(sparse_0, sparse_1, sparse_2, sparse_3, sparse_4, sparse_5, sparse_6, sparse_7, sparse_8, sparse_9, sparse_10, sparse_11, sparse_12, sparse_13, sparse_14, sparse_15, sparse_16, sparse_17, sparse_18, sparse_19, sparse_20, sparse_21, sparse_22, sparse_23, sparse_24, sparse_25, dense_0, dense_1, dense_2, dense_3, dense_4, dense_5, dense_6, dense_7, dense_8, dense_9, dense_10, dense_11, dense_12, emb_0, emb_1, emb_2, emb_3, emb_4, emb_5, emb_6, emb_7, emb_8, emb_9, emb_10, emb_11, emb_12, emb_13, emb_14, emb_15, emb_16, emb_17, emb_18, emb_19, emb_20, emb_21, emb_22, emb_23, emb_24, emb_25, fc_W, fc_b):
    sparse = [sparse_0, sparse_1, sparse_2, sparse_3, sparse_4, sparse_5, sparse_6, sparse_7, sparse_8, sparse_9, sparse_10, sparse_11, sparse_12, sparse_13, sparse_14, sparse_15, sparse_16, sparse_17, sparse_18, sparse_19, sparse_20, sparse_21, sparse_22, sparse_23, sparse_24, sparse_25]
    dense = [dense_0, dense_1, dense_2, dense_3, dense_4, dense_5, dense_6, dense_7, dense_8, dense_9, dense_10, dense_11, dense_12]
    tables = [emb_0, emb_1, emb_2, emb_3, emb_4, emb_5, emb_6, emb_7, emb_8, emb_9, emb_10, emb_11, emb_12, emb_13, emb_14, emb_15, emb_16, emb_17, emb_18, emb_19, emb_20, emb_21, emb_22, emb_23, emb_24, emb_25]

    idx = [s.astype(jnp.int32).reshape(1, _BATCH) for s in sparse]
    dns = [d.astype(jnp.float32).reshape(1, _BATCH) for d in dense]
    w = jnp.concatenate([fc_W.reshape(-1), fc_b.reshape(-1)]).astype(jnp.float32)
    wrep = jnp.broadcast_to(w[:, None], (_NSF + _NDF + 1, _L))
    wide_tabs = [t.reshape(1, -1) for t in tables]

    out = _sc_call(*idx, *dns, wrep, *wide_tabs)
    return out.reshape(_BATCH, 1)
